# baseline (speedup 1.0000x reference)
import functools

import jax
import jax.numpy as jnp
from jax import lax
from jax.experimental import pallas as pl
from jax.experimental.pallas import tpu as pltpu
from jax.experimental.pallas import tpu_sc as plsc

_B = 4096
_S = 200
_D = 64
_V = 1000000

_NC = 2
_NS = 16
_NW = _NC * _NS
_C = _B // _NW
_H = _D // 2

_mesh = plsc.VectorSubcoreMesh(core_axis_name="c", subcore_axis_name="s")


@functools.partial(
    pl.kernel,
    out_type=jax.ShapeDtypeStruct((_S, _D // 8, _B // _C, 8, _C), jnp.float32),
    mesh=_mesh,
    compiler_params=pltpu.CompilerParams(
        needs_layout_passes=False, use_tc_tiling_on_sc=False),
    scratch_types=[
        pltpu.VMEM((_S // 8, 8, _C), jnp.int32),
        pltpu.VMEM((2, _C), jnp.int32),
        pltpu.VMEM((2, _C), jnp.int32),
        pltpu.VMEM((_C, _H), jnp.float32),
        pltpu.VMEM((_C, _H), jnp.float32),
        pltpu.VMEM((_C, _H), jnp.float32),
        pltpu.VMEM((_C, _H), jnp.float32),
        pltpu.VMEM((_D, _C + 1), jnp.float32),
        pltpu.VMEM((_D, _C + 1), jnp.float32),
        pltpu.VMEM((_S * _D,), jnp.float32),
        pltpu.VMEM((_C,), jnp.float32),
        pltpu.SemaphoreType.DMA((2,)),
        pltpu.SemaphoreType.DMA((2,)),
    ],
)
def _embed(ids_hbm, tok_hbm, pos_hbm, out_hbm,
           ids_v, ridx_v, oidx_v, even0_v, even1_v, odd0_v, odd1_v,
           stage0_v, stage1_v, pos_v, msk_v, gsem, osem):
    wid = lax.axis_index("s") * _NC + lax.axis_index("c")
    pltpu.sync_copy(pos_hbm, pos_v)
    for k in range(_S // 8):
        pltpu.sync_copy(ids_hbm.at[k, wid], ids_v.at[k])
    lane = jnp.arange(16, dtype=jnp.int32)
    evens = (even0_v, even1_v)
    odds = (odd0_v, odd1_v)
    stages = (stage0_v, stage1_v)

    def fire_gather(s, j):
        sh = lax.div(s, 8)
        sl = lax.rem(s, 8)

        @plsc.parallel_loop(0, _C // 16)
        def _expand(t):
            tv = ids_v[sh, sl, pl.ds(t * 16, 16)]
            tv3 = tv * 3
            ridx_v[j, pl.ds(t * 16, 16)] = tv3
            oidx_v[j, pl.ds(t * 16, 16)] = tv3 + 1

        pltpu.async_copy(tok_hbm.at[ridx_v.at[j]], evens[j], gsem.at[j])
        pltpu.async_copy(tok_hbm.at[oidx_v.at[j]], odds[j], gsem.at[j])

    def wait_gather(j):
        pltpu.make_async_copy(tok_hbm.at[ridx_v.at[j]], evens[j],
                              gsem.at[j]).wait()
        pltpu.make_async_copy(tok_hbm.at[oidx_v.at[j]], odds[j],
                              gsem.at[j]).wait()

    def fire_out(s, j):
        for dh in range(_D // 8):
            pltpu.async_copy(stages[j].at[pl.ds(dh * 8, 8), pl.ds(0, _C)],
                             out_hbm.at[s, dh, wid], osem.at[j])

    def wait_out(s, j):
        for dh in range(_D // 8):
            pltpu.make_async_copy(stages[j].at[pl.ds(dh * 8, 8), pl.ds(0, _C)],
                                  out_hbm.at[s, dh, wid], osem.at[j]).wait()

    def compute(s, j):
        sh = lax.div(s, 8)
        sl = lax.rem(s, 8)
        wait_gather(j)
        even_v, odd_v, stage_v = evens[j], odds[j], stages[j]
        for t in range(_C // 16):
            tv = ids_v[sh, sl, pl.ds(t * 16, 16)]
            msk_v[pl.ds(t * 16, 16)] = jnp.where(tv != 0, 1.0, 0.0)
        s64 = s * _D
        pvecs = [pos_v[pl.ds(s64 + k * 16, 16)] for k in range(_D // 16)]
        dvecs = [lane + (k * 16) for k in range(_D // 16)]

        @plsc.parallel_loop(0, _C, unroll=2)
        def _row(b):
            m = plsc.load_gather(msk_v, [jnp.full((16,), b, jnp.int32)])
            cb = jnp.full((16,), b, jnp.int32)
            for k in range(_D // 16):
                if k < 2:
                    v = even_v[b, pl.ds(k * 16, 16)]
                else:
                    v = odd_v[b, pl.ds((k - 2) * 16, 16)]
                plsc.store_scatter(stage_v, [dvecs[k], cb],
                                   (v * 8.0 + pvecs[k]) * m)

    fire_gather(0, 0)

    def body(s2, carry):
        s = s2 * 2
        fire_gather(s + 1, 1)

        @pl.when(s2 > 0)
        def _():
            wait_out(s - 2, 0)
        compute(s, 0)
        fire_out(s, 0)

        @pl.when(s2 < _S // 2 - 1)
        def _():
            fire_gather(s + 2, 0)

        @pl.when(s2 > 0)
        def _():
            wait_out(s - 1, 1)
        compute(s + 1, 1)
        fire_out(s + 1, 1)
        return carry

    lax.fori_loop(0, _S // 2, body, 0)
    wait_out(_S - 2, 0)
    wait_out(_S - 1, 1)


def kernel(inputs, token_table, pos_table):
    ids4 = (inputs.T.reshape(_S // 8, 8, _B // _C, _C)
            .transpose(0, 2, 1, 3))
    tok_pad = jnp.pad(token_table, ((0, 0), (0, _H)))
    tok3 = tok_pad.reshape(3 * _V, _H)
    out5 = _embed(ids4, tok3, pos_table.reshape(_S * _D))
    return (out5.transpose(2, 4, 0, 1, 3)
            .reshape(_B, _S, _D))


# speedup vs baseline: 1.7187x; 1.7187x over previous
import functools

import jax
import jax.numpy as jnp
from jax import lax
from jax.experimental import pallas as pl
from jax.experimental.pallas import tpu as pltpu
from jax.experimental.pallas import tpu_sc as plsc

_B = 4096
_S = 200
_D = 64
_V = 1000000

_NC = 2
_NS = 16
_NW = _NC * _NS
_C = _B // _NW
_H = _D // 2

_mesh = plsc.VectorSubcoreMesh(core_axis_name="c", subcore_axis_name="s")


@functools.partial(
    pl.kernel,
    out_type=jax.ShapeDtypeStruct((_S, _D // 8, _B // _C, 8, _C), jnp.float32),
    mesh=_mesh,
    compiler_params=pltpu.CompilerParams(
        needs_layout_passes=False, use_tc_tiling_on_sc=False),
    scratch_types=[
        pltpu.VMEM((_S // 8, 8, _C), jnp.int32),
        pltpu.VMEM((2, _C), jnp.int32),
        pltpu.VMEM((_C, _D), jnp.float32),
        pltpu.VMEM((_C, _D), jnp.float32),
        pltpu.VMEM((_D, _C + 1), jnp.float32),
        pltpu.VMEM((_D, _C + 1), jnp.float32),
        pltpu.VMEM((_S * _D,), jnp.float32),
        pltpu.VMEM((_C,), jnp.float32),
        pltpu.SemaphoreType.DMA((2,)),
        pltpu.SemaphoreType.DMA((2,)),
    ],
)
def _embed(ids_hbm, tok_hbm, pos_hbm, out_hbm,
           ids_v, ridx_v, rows0_v, rows1_v, stage0_v, stage1_v, pos_v, msk_v,
           gsem, osem):
    wid = lax.axis_index("s") * _NC + lax.axis_index("c")
    pltpu.sync_copy(pos_hbm, pos_v)
    for k in range(_S // 8):
        pltpu.sync_copy(ids_hbm.at[k, wid], ids_v.at[k])
    lane = jnp.arange(16, dtype=jnp.int32)
    rows = (rows0_v, rows1_v)
    stages = (stage0_v, stage1_v)

    def fire_gather(s, j):
        sh = lax.div(s, 8)
        sl = lax.rem(s, 8)

        @plsc.parallel_loop(0, _C // 16)
        def _expand(t):
            tv = ids_v[sh, sl, pl.ds(t * 16, 16)]
            ridx_v[j, pl.ds(t * 16, 16)] = tv + tv

        pltpu.async_copy(tok_hbm.at[ridx_v.at[j]], rows[j], gsem.at[j])

    def wait_gather(j):
        pltpu.make_async_copy(tok_hbm.at[ridx_v.at[j]], rows[j],
                              gsem.at[j]).wait()

    def fire_out(s, j):
        for dh in range(_D // 8):
            pltpu.async_copy(stages[j].at[pl.ds(dh * 8, 8), pl.ds(0, _C)],
                             out_hbm.at[s, dh, wid], osem.at[j])

    def wait_out(s, j):
        for dh in range(_D // 8):
            pltpu.make_async_copy(stages[j].at[pl.ds(dh * 8, 8), pl.ds(0, _C)],
                                  out_hbm.at[s, dh, wid], osem.at[j]).wait()

    def compute(s, j):
        sh = lax.div(s, 8)
        sl = lax.rem(s, 8)
        wait_gather(j)
        rows_v, stage_v = rows[j], stages[j]
        for t in range(_C // 16):
            tv = ids_v[sh, sl, pl.ds(t * 16, 16)]
            msk_v[pl.ds(t * 16, 16)] = jnp.where(tv != 0, 1.0, 0.0)
        s64 = s * _D
        pvecs = [pos_v[pl.ds(s64 + k * 16, 16)] for k in range(_D // 16)]
        dvecs = [lane + (k * 16) for k in range(_D // 16)]

        @plsc.parallel_loop(0, _C, unroll=4)
        def _row(b):
            m = plsc.load_gather(msk_v, [jnp.full((16,), b, jnp.int32)])
            cb = jnp.full((16,), b, jnp.int32)
            for k in range(_D // 16):
                v = rows_v[b, pl.ds(k * 16, 16)]
                plsc.store_scatter(stage_v, [dvecs[k], cb],
                                   (v * 8.0 + pvecs[k]) * m)

    fire_gather(0, 0)

    def body(s2, carry):
        s = s2 * 2
        fire_gather(s + 1, 1)

        @pl.when(s2 > 0)
        def _():
            wait_out(s - 2, 0)
        compute(s, 0)
        fire_out(s, 0)

        @pl.when(s2 < _S // 2 - 1)
        def _():
            fire_gather(s + 2, 0)

        @pl.when(s2 > 0)
        def _():
            wait_out(s - 1, 1)
        compute(s + 1, 1)
        fire_out(s + 1, 1)
        return carry

    lax.fori_loop(0, _S // 2, body, 0)
    wait_out(_S - 2, 0)
    wait_out(_S - 1, 1)


def kernel(inputs, token_table, pos_table):
    ids4 = (inputs.T.reshape(_S // 8, 8, _B // _C, _C)
            .transpose(0, 2, 1, 3))
    tok_pad = jnp.pad(token_table, ((0, 0), (0, _D)))
    tok2 = tok_pad.reshape(2 * _V, _D)
    out5 = _embed(ids4, tok2, pos_table.reshape(_S * _D))
    return (out5.transpose(2, 4, 0, 1, 3)
            .reshape(_B, _S, _D))
